# baseline (speedup 1.0000x reference)
import functools
import math

import jax
import jax.numpy as jnp
from jax import lax
from jax.experimental import pallas as pl
from jax.experimental.pallas import tpu as pltpu
from jax.experimental.pallas import tpu_sc as plsc

MODEL_DIM = 64
LANES = 16
NUM_CORES = 2
NUM_SUBCORES = 16
NW = NUM_CORES * NUM_SUBCORES
SPLITS = (0, 96, 200)
NBUF = 4
HALF = 2
SCALE = 8.0


def _make_emb_kernel(S: int, T: int, D: int):
    assert S % NW == 0
    spw = S // NW
    assert spw % NBUF == 0 and spw // NBUF >= 2

    mesh = plsc.VectorSubcoreMesh(core_axis_name="c", subcore_axis_name="s")

    @functools.partial(
        pl.kernel,
        mesh=mesh,
        out_type=jax.ShapeDtypeStruct((S, T, D), jnp.float32),
        compiler_params=pltpu.CompilerParams(use_tc_tiling_on_sc=False),
        scratch_types=[
            pltpu.VMEM((spw, T), jnp.int32),
            pltpu.VMEM((NBUF, T, D), jnp.float32),
            pltpu.SemaphoreType.DMA((NBUF,)),
            pltpu.SemaphoreType.DMA((NBUF,)),
        ],
    )
    def emb(table_hbm, idx_hbm, out_hbm, idx_v, rows_v, gsem, wsem):
        wid = lax.axis_index("s") * NUM_CORES + lax.axis_index("c")
        sent0 = wid * spw
        pltpu.sync_copy(idx_hbm.at[pl.ds(sent0, spw)], idx_v)

        def gather_refs(s, b):
            for lo, hi in zip(SPLITS[:-1], SPLITS[1:]):
                yield (table_hbm.at[idx_v.at[s, pl.ds(lo, hi - lo)]],
                       rows_v.at[b, pl.ds(lo, hi - lo)], gsem.at[b])

        def issue_gathers(s, b):
            for refs in gather_refs(s, b):
                pltpu.async_copy(*refs)

        def wait_gathers(s, b):
            for refs in gather_refs(s, b):
                pltpu.make_async_copy(*refs).wait()

        def wait_writeback(b):
            pltpu.make_async_copy(rows_v.at[b], out_hbm.at[sent0],
                                  wsem.at[b]).wait()

        def scale_block(b):
            @plsc.parallel_loop(0, T, unroll=4)
            def _(r):
                for k in range(D // LANES):
                    sl = pl.ds(k * LANES, LANES)
                    rows_v[b, r, sl] = rows_v[b, r, sl] * SCALE

        def process(s, b):
            wait_gathers(s, b)
            scale_block(b)
            pltpu.async_copy(rows_v.at[b], out_hbm.at[sent0 + s], wsem.at[b])

        for q in range(HALF):
            issue_gathers(q, q)

        for b in range(NBUF):
            q = b + HALF
            if q >= NBUF:
                wait_writeback(q % NBUF)
            issue_gathers(q, q % NBUF)
            process(b, b)

        def outer(go, carry):
            g0 = go * NBUF
            for b in range(NBUF):
                qb = (b + HALF) % NBUF
                wait_writeback(qb)
                issue_gathers(g0 + b + HALF, qb)
                process(g0 + b, b)
            return carry

        lax.fori_loop(1, spw // NBUF - 1, outer, 0)

        gl = spw - NBUF
        for b in range(HALF):
            qb = (b + HALF) % NBUF
            wait_writeback(qb)
            issue_gathers(gl + b + HALF, qb)
            process(gl + b, b)
        for b in range(HALF, NBUF):
            process(gl + b, b)

        for b in range(NBUF):
            wait_writeback(b)

    return emb


def kernel(x, table):
    S, T = x.shape
    D = table.shape[1]
    return _make_emb_kernel(S, T, D)(table, x.astype(jnp.int32))
